# baseline (speedup 1.0000x reference)
import functools

import jax
import jax.numpy as jnp
from jax import lax
from jax.experimental import pallas as pl
from jax.experimental.pallas import tpu as pltpu
from jax.experimental.pallas import tpu_sc as plsc

_K = 4
_NBUF = 2
_CH = 16
_NSLOT = 8


def _sc_gather_kernel(nc, bpw, nchunk, table_ref, idx_ref, out_ref, idx_v,
                      rows_v, gsems, wsems):
    w = lax.axis_index("s") * nc + lax.axis_index("c")
    base = w * bpw
    pltpu.sync_copy(idx_ref.at[w], idx_v)

    for b in range(_NBUF):
        pltpu.make_async_copy(
            table_ref.at[idx_v.at[b]], rows_v.at[b], gsems.at[b]
        ).start()

    @pl.loop(0, nchunk, step=_NBUF)
    def _chunks(c):
        for b in range(_NBUF):
            cc = c + b
            pltpu.make_async_copy(
                table_ref.at[idx_v.at[cc]], rows_v.at[b], gsems.at[b]
            ).wait()
            pltpu.make_async_copy(
                rows_v.at[b], out_ref.at[pl.ds(base + cc * _K, _K)],
                wsems.at[b],
            ).start()

            @pl.when(cc + _NBUF < nchunk)
            def _():
                pltpu.make_async_copy(
                    rows_v.at[b], out_ref.at[pl.ds(base + cc * _K, _K)],
                    wsems.at[b],
                ).wait()
                pltpu.make_async_copy(
                    table_ref.at[idx_v.at[cc + _NBUF]], rows_v.at[b],
                    gsems.at[b]
                ).start()

    for b in range(_NBUF):
        cc = nchunk - _NBUF + b
        pltpu.make_async_copy(
            rows_v.at[b], out_ref.at[pl.ds(base + cc * _K, _K)], wsems.at[b]
        ).wait()


def _tc_ce_kernel(n_steps, idx_ref, table_ref, tgt_ref, acc_ref, rows, sems):
    i = pl.program_id(0)
    s = lax.rem(i, _NSLOT)

    @pl.when(i == 0)
    def _prime():
        for q in range(_NSLOT):
            for j in range(_CH):
                pltpu.make_async_copy(
                    table_ref.at[idx_ref[q * _CH + j]], rows.at[q, j],
                    sems.at[q, j],
                ).start()

    for j in range(_CH):
        pltpu.make_async_copy(
            table_ref.at[idx_ref[i * _CH + j]], rows.at[s, j], sems.at[s, j]
        ).wait()

    block = rows[s]
    m = jnp.max(block, axis=1, keepdims=True)
    e = jnp.sum(jnp.exp(block - m), axis=1, keepdims=True)
    logz = m + jnp.log(e)
    lanes = lax.broadcasted_iota(jnp.int32, block.shape, 1)
    onehot = lanes == tgt_ref[...]
    picked = jnp.sum(jnp.where(onehot, block, 0.0), axis=1, keepdims=True)
    part = jnp.sum(logz - picked)

    @pl.when(i == 0)
    def _init():
        acc_ref[...] = jnp.zeros((1, 1), jnp.float32)

    acc_ref[...] += part

    @pl.when(i + _NSLOT < n_steps)
    def _next():
        for j in range(_CH):
            pltpu.make_async_copy(
                table_ref.at[idx_ref[(i + _NSLOT) * _CH + j]], rows.at[s, j],
                sems.at[s, j],
            ).start()


def kernel(idx, targets, table):
    B, T = idx.shape
    V, C = table.shape
    n_tok = B * T
    idx_flat = idx.reshape(n_tok).astype(jnp.int32)
    tgt_flat = targets.reshape(n_tok).astype(jnp.int32)

    info = plsc.get_sparse_core_info()
    nc, ns = info.num_cores, info.num_subcores
    nw = nc * ns
    bpw = n_tok // nw
    nchunk = bpw // _K

    idx3d = idx_flat.reshape(nw, nchunk, _K)

    sc_call = pl.kernel(
        functools.partial(_sc_gather_kernel, nc, bpw, nchunk),
        out_type=jax.ShapeDtypeStruct((n_tok, C), jnp.float32),
        mesh=plsc.VectorSubcoreMesh(
            core_axis_name="c", subcore_axis_name="s"
        ),
        scratch_types=[
            pltpu.VMEM((nchunk, _K), jnp.int32),
            pltpu.VMEM((_NBUF, _K, C), jnp.float32),
            pltpu.SemaphoreType.DMA((_NBUF,)),
            pltpu.SemaphoreType.DMA((_NBUF,)),
        ],
    )
    logits_flat = sc_call(table, idx3d)

    n_steps = n_tok // _CH
    grid_spec = pltpu.PrefetchScalarGridSpec(
        num_scalar_prefetch=1,
        grid=(n_steps,),
        in_specs=[
            pl.BlockSpec(memory_space=pltpu.HBM),
            pl.BlockSpec((_CH, 1), lambda i, idx_ref: (i, 0)),
        ],
        out_specs=pl.BlockSpec((1, 1), lambda i, idx_ref: (0, 0)),
        scratch_shapes=[
            pltpu.VMEM((_NSLOT, _CH, C), jnp.float32),
            pltpu.SemaphoreType.DMA((_NSLOT, _CH)),
        ],
    )
    loss_sum = pl.pallas_call(
        functools.partial(_tc_ce_kernel, n_steps),
        grid_spec=grid_spec,
        out_shape=jax.ShapeDtypeStruct((1, 1), jnp.float32),
    )(idx_flat, table, tgt_flat.reshape(n_tok, 1))

    loss = loss_sum[0, 0] / n_tok
    return logits_flat.reshape(B, T, C), loss


# speedup vs baseline: 1.4519x; 1.4519x over previous
import functools

import jax
import jax.numpy as jnp
from jax import lax
from jax.experimental import pallas as pl
from jax.experimental.pallas import tpu as pltpu
from jax.experimental.pallas import tpu_sc as plsc

_K = 4
_NBUF = 2
_CH = 16
_NSLOT = 8
_NS1 = 1536
_BLK = 256


def _sc_gather_kernel(nc, bpw, nchunk, table_ref, idx_ref, out_ref, idx_v,
                      rows_v, gsems, wsems):
    w = lax.axis_index("s") * nc + lax.axis_index("c")
    base = w * bpw
    pltpu.sync_copy(idx_ref.at[w], idx_v)

    for b in range(_NBUF):
        pltpu.make_async_copy(
            table_ref.at[idx_v.at[b]], rows_v.at[b], gsems.at[b]
        ).start()

    @pl.loop(0, nchunk, step=_NBUF)
    def _chunks(c):
        for b in range(_NBUF):
            cc = c + b
            pltpu.make_async_copy(
                table_ref.at[idx_v.at[cc]], rows_v.at[b], gsems.at[b]
            ).wait()
            pltpu.make_async_copy(
                rows_v.at[b], out_ref.at[pl.ds(base + cc * _K, _K)],
                wsems.at[b],
            ).start()

            @pl.when(cc + _NBUF < nchunk)
            def _():
                pltpu.make_async_copy(
                    rows_v.at[b], out_ref.at[pl.ds(base + cc * _K, _K)],
                    wsems.at[b],
                ).wait()
                pltpu.make_async_copy(
                    table_ref.at[idx_v.at[cc + _NBUF]], rows_v.at[b],
                    gsems.at[b]
                ).start()

    for b in range(_NBUF):
        cc = nchunk - _NBUF + b
        pltpu.make_async_copy(
            rows_v.at[b], out_ref.at[pl.ds(base + cc * _K, _K)], wsems.at[b]
        ).wait()


def _tc_ce_kernel(n_steps, idx_ref, table_ref, tgt_ref, acc_ref, rows, sems):
    i = pl.program_id(0)
    s = lax.rem(i, _NSLOT)

    @pl.when(i == 0)
    def _prime():
        for q in range(_NSLOT):
            for j in range(_CH):
                pltpu.make_async_copy(
                    table_ref.at[idx_ref[q * _CH + j]], rows.at[q, j],
                    sems.at[q, j],
                ).start()

    for j in range(_CH):
        pltpu.make_async_copy(
            table_ref.at[idx_ref[i * _CH + j]], rows.at[s, j], sems.at[s, j]
        ).wait()

    block = rows[s]
    m = jnp.max(block, axis=1, keepdims=True)
    e = jnp.sum(jnp.exp(block - m), axis=1, keepdims=True)
    logz = m + jnp.log(e)
    lanes = lax.broadcasted_iota(jnp.int32, block.shape, 1)
    onehot = lanes == tgt_ref[...]
    picked = jnp.sum(jnp.where(onehot, block, 0.0), axis=1, keepdims=True)
    part = jnp.sum(logz - picked)

    @pl.when(i == 0)
    def _init():
        acc_ref[...] = jnp.zeros((1, 1), jnp.float32)

    acc_ref[...] += part

    @pl.when(i + _NSLOT < n_steps)
    def _next():
        for j in range(_CH):
            pltpu.make_async_copy(
                table_ref.at[idx_ref[(i + _NSLOT) * _CH + j]], rows.at[s, j],
                sems.at[s, j],
            ).start()


def _tc_stream_kernel(logits_ref, tgt_ref, acc_ref):
    i = pl.program_id(0)
    block = logits_ref[...]
    m = jnp.max(block, axis=1, keepdims=True)
    e = jnp.sum(jnp.exp(block - m), axis=1, keepdims=True)
    logz = m + jnp.log(e)
    lanes = lax.broadcasted_iota(jnp.int32, block.shape, 1)
    onehot = lanes == tgt_ref[...]
    picked = jnp.sum(jnp.where(onehot, block, 0.0), axis=1, keepdims=True)
    part = jnp.sum(logz - picked)

    @pl.when(i == 0)
    def _init():
        acc_ref[...] = jnp.zeros((1, 1), jnp.float32)

    acc_ref[...] += part


def kernel(idx, targets, table):
    B, T = idx.shape
    V, C = table.shape
    n_tok = B * T
    idx_flat = idx.reshape(n_tok).astype(jnp.int32)
    tgt_flat = targets.reshape(n_tok).astype(jnp.int32)

    info = plsc.get_sparse_core_info()
    nc, ns = info.num_cores, info.num_subcores
    nw = nc * ns
    bpw = n_tok // nw
    nchunk = bpw // _K

    idx3d = idx_flat.reshape(nw, nchunk, _K)

    sc_call = pl.kernel(
        functools.partial(_sc_gather_kernel, nc, bpw, nchunk),
        out_type=jax.ShapeDtypeStruct((n_tok, C), jnp.float32),
        mesh=plsc.VectorSubcoreMesh(
            core_axis_name="c", subcore_axis_name="s"
        ),
        scratch_types=[
            pltpu.VMEM((nchunk, _K), jnp.int32),
            pltpu.VMEM((_NBUF, _K, C), jnp.float32),
            pltpu.SemaphoreType.DMA((_NBUF,)),
            pltpu.SemaphoreType.DMA((_NBUF,)),
        ],
    )
    logits_flat = sc_call(table, idx3d)

    tgt2d = tgt_flat.reshape(n_tok, 1)

    n_steps = _NS1 // _CH
    grid_spec = pltpu.PrefetchScalarGridSpec(
        num_scalar_prefetch=1,
        grid=(n_steps,),
        in_specs=[
            pl.BlockSpec(memory_space=pltpu.HBM),
            pl.BlockSpec((_CH, 1), lambda i, idx_ref: (i, 0)),
        ],
        out_specs=pl.BlockSpec((1, 1), lambda i, idx_ref: (0, 0)),
        scratch_shapes=[
            pltpu.VMEM((_NSLOT, _CH, C), jnp.float32),
            pltpu.SemaphoreType.DMA((_NSLOT, _CH)),
        ],
    )
    loss_sum1 = pl.pallas_call(
        functools.partial(_tc_ce_kernel, n_steps),
        grid_spec=grid_spec,
        out_shape=jax.ShapeDtypeStruct((1, 1), jnp.float32),
    )(idx_flat, table, tgt2d)

    blk0 = _NS1 // _BLK
    loss_sum2 = pl.pallas_call(
        _tc_stream_kernel,
        grid=((n_tok - _NS1) // _BLK,),
        in_specs=[
            pl.BlockSpec((_BLK, C), lambda i: (i + blk0, 0)),
            pl.BlockSpec((_BLK, 1), lambda i: (i + blk0, 0)),
        ],
        out_specs=pl.BlockSpec((1, 1), lambda i: (0, 0)),
        out_shape=jax.ShapeDtypeStruct((1, 1), jnp.float32),
    )(logits_flat, tgt2d)

    loss = (loss_sum1[0, 0] + loss_sum2[0, 0]) / n_tok
    return logits_flat.reshape(B, T, C), loss


# speedup vs baseline: 1.5241x; 1.0497x over previous
import functools

import jax
import jax.numpy as jnp
from jax import lax
from jax.experimental import pallas as pl
from jax.experimental.pallas import tpu as pltpu
from jax.experimental.pallas import tpu_sc as plsc

_K = 4
_NBUF = 2
_CH = 16
_NSLOT = 8
_NS1 = 1280
_BLK = 256


def _sc_gather_kernel(nc, bpw, nchunk, table_ref, idx_ref, out_ref, idx_v,
                      rows_v, gsems, wsems):
    w = lax.axis_index("s") * nc + lax.axis_index("c")
    base = w * bpw
    pltpu.sync_copy(idx_ref.at[w], idx_v)

    for b in range(_NBUF):
        pltpu.make_async_copy(
            table_ref.at[idx_v.at[b]], rows_v.at[b], gsems.at[b]
        ).start()

    @pl.loop(0, nchunk, step=_NBUF)
    def _chunks(c):
        for b in range(_NBUF):
            cc = c + b
            pltpu.make_async_copy(
                table_ref.at[idx_v.at[cc]], rows_v.at[b], gsems.at[b]
            ).wait()
            pltpu.make_async_copy(
                rows_v.at[b], out_ref.at[pl.ds(base + cc * _K, _K)],
                wsems.at[b],
            ).start()

            @pl.when(cc + _NBUF < nchunk)
            def _():
                pltpu.make_async_copy(
                    rows_v.at[b], out_ref.at[pl.ds(base + cc * _K, _K)],
                    wsems.at[b],
                ).wait()
                pltpu.make_async_copy(
                    table_ref.at[idx_v.at[cc + _NBUF]], rows_v.at[b],
                    gsems.at[b]
                ).start()

    for b in range(_NBUF):
        cc = nchunk - _NBUF + b
        pltpu.make_async_copy(
            rows_v.at[b], out_ref.at[pl.ds(base + cc * _K, _K)], wsems.at[b]
        ).wait()


def _tc_ce_kernel(n_steps, idx_ref, table_ref, tgt_ref, acc_ref, rows, sems):
    i = pl.program_id(0)
    s = lax.rem(i, _NSLOT)

    @pl.when(i == 0)
    def _prime():
        for q in range(_NSLOT):
            for j in range(_CH):
                pltpu.make_async_copy(
                    table_ref.at[idx_ref[q * _CH + j]], rows.at[q, j],
                    sems.at[q, j],
                ).start()

    for j in range(_CH):
        pltpu.make_async_copy(
            table_ref.at[idx_ref[i * _CH + j]], rows.at[s, j], sems.at[s, j]
        ).wait()

    block = rows[s]
    m = jnp.max(block, axis=1, keepdims=True)
    e = jnp.sum(jnp.exp(block - m), axis=1, keepdims=True)
    logz = m + jnp.log(e)
    lanes = lax.broadcasted_iota(jnp.int32, block.shape, 1)
    onehot = lanes == tgt_ref[...]
    picked = jnp.sum(jnp.where(onehot, block, 0.0), axis=1, keepdims=True)
    part = jnp.sum(logz - picked)

    @pl.when(i == 0)
    def _init():
        acc_ref[...] = jnp.zeros((1, 1), jnp.float32)

    acc_ref[...] += part

    @pl.when(i + _NSLOT < n_steps)
    def _next():
        for j in range(_CH):
            pltpu.make_async_copy(
                table_ref.at[idx_ref[(i + _NSLOT) * _CH + j]], rows.at[s, j],
                sems.at[s, j],
            ).start()


def _tc_stream_kernel(logits_ref, tgt_ref, acc_ref):
    i = pl.program_id(0)
    block = logits_ref[...]
    m = jnp.max(block, axis=1, keepdims=True)
    e = jnp.sum(jnp.exp(block - m), axis=1, keepdims=True)
    logz = m + jnp.log(e)
    lanes = lax.broadcasted_iota(jnp.int32, block.shape, 1)
    onehot = lanes == tgt_ref[...]
    picked = jnp.sum(jnp.where(onehot, block, 0.0), axis=1, keepdims=True)
    part = jnp.sum(logz - picked)

    @pl.when(i == 0)
    def _init():
        acc_ref[...] = jnp.zeros((1, 1), jnp.float32)

    acc_ref[...] += part


def kernel(idx, targets, table):
    B, T = idx.shape
    V, C = table.shape
    n_tok = B * T
    idx_flat = idx.reshape(n_tok).astype(jnp.int32)
    tgt_flat = targets.reshape(n_tok).astype(jnp.int32)

    info = plsc.get_sparse_core_info()
    nc, ns = info.num_cores, info.num_subcores
    nw = nc * ns
    bpw = n_tok // nw
    nchunk = bpw // _K

    idx3d = idx_flat.reshape(nw, nchunk, _K)

    sc_call = pl.kernel(
        functools.partial(_sc_gather_kernel, nc, bpw, nchunk),
        out_type=jax.ShapeDtypeStruct((n_tok, C), jnp.float32),
        mesh=plsc.VectorSubcoreMesh(
            core_axis_name="c", subcore_axis_name="s"
        ),
        scratch_types=[
            pltpu.VMEM((nchunk, _K), jnp.int32),
            pltpu.VMEM((_NBUF, _K, C), jnp.float32),
            pltpu.SemaphoreType.DMA((_NBUF,)),
            pltpu.SemaphoreType.DMA((_NBUF,)),
        ],
    )
    logits_flat = sc_call(table, idx3d)

    tgt2d = tgt_flat.reshape(n_tok, 1)

    n_steps = _NS1 // _CH
    grid_spec = pltpu.PrefetchScalarGridSpec(
        num_scalar_prefetch=1,
        grid=(n_steps,),
        in_specs=[
            pl.BlockSpec(memory_space=pltpu.HBM),
            pl.BlockSpec((_CH, 1), lambda i, idx_ref: (i, 0)),
        ],
        out_specs=pl.BlockSpec((1, 1), lambda i, idx_ref: (0, 0)),
        scratch_shapes=[
            pltpu.VMEM((_NSLOT, _CH, C), jnp.float32),
            pltpu.SemaphoreType.DMA((_NSLOT, _CH)),
        ],
    )
    loss_sum1 = pl.pallas_call(
        functools.partial(_tc_ce_kernel, n_steps),
        grid_spec=grid_spec,
        out_shape=jax.ShapeDtypeStruct((1, 1), jnp.float32),
    )(idx_flat, table, tgt2d)

    blk0 = _NS1 // _BLK
    loss_sum2 = pl.pallas_call(
        _tc_stream_kernel,
        grid=((n_tok - _NS1) // _BLK,),
        in_specs=[
            pl.BlockSpec((_BLK, C), lambda i: (i + blk0, 0)),
            pl.BlockSpec((_BLK, 1), lambda i: (i + blk0, 0)),
        ],
        out_specs=pl.BlockSpec((1, 1), lambda i: (0, 0)),
        out_shape=jax.ShapeDtypeStruct((1, 1), jnp.float32),
    )(logits_flat, tgt2d)

    loss = (loss_sum1[0, 0] + loss_sum2[0, 0]) / n_tok
    return logits_flat.reshape(B, T, C), loss


# speedup vs baseline: 1.5569x; 1.0216x over previous
import functools

import jax
import jax.numpy as jnp
from jax import lax
from jax.experimental import pallas as pl
from jax.experimental.pallas import tpu as pltpu
from jax.experimental.pallas import tpu_sc as plsc

_K = 2
_NBUF = 4
_BLK = 512


def _sc_gather_kernel(nc, bpw, nchunk, table_ref, idx_ref, out_ref, idx_v,
                      rows_v, gsems, wsems):
    w = lax.axis_index("s") * nc + lax.axis_index("c")
    base = w * bpw
    pltpu.sync_copy(idx_ref.at[w], idx_v)

    for b in range(_NBUF):
        pltpu.make_async_copy(
            table_ref.at[idx_v.at[b]], rows_v.at[b], gsems.at[b]
        ).start()

    @pl.loop(0, nchunk, step=_NBUF)
    def _chunks(c):
        for b in range(_NBUF):
            cc = c + b
            pltpu.make_async_copy(
                table_ref.at[idx_v.at[cc]], rows_v.at[b], gsems.at[b]
            ).wait()
            pltpu.make_async_copy(
                rows_v.at[b], out_ref.at[pl.ds(base + cc * _K, _K)],
                wsems.at[b],
            ).start()

            @pl.when(cc + _NBUF < nchunk)
            def _():
                pltpu.make_async_copy(
                    rows_v.at[b], out_ref.at[pl.ds(base + cc * _K, _K)],
                    wsems.at[b],
                ).wait()
                pltpu.make_async_copy(
                    table_ref.at[idx_v.at[cc + _NBUF]], rows_v.at[b],
                    gsems.at[b]
                ).start()

    for b in range(_NBUF):
        cc = nchunk - _NBUF + b
        pltpu.make_async_copy(
            rows_v.at[b], out_ref.at[pl.ds(base + cc * _K, _K)], wsems.at[b]
        ).wait()


def _tc_loss_kernel(logits_ref, tgt_ref, acc_ref):
    i = pl.program_id(0)
    block = logits_ref[...]
    m = jnp.max(block, axis=1, keepdims=True)
    e = jnp.sum(jnp.exp(block - m), axis=1, keepdims=True)
    logz = m + jnp.log(e)
    lanes = lax.broadcasted_iota(jnp.int32, block.shape, 1)
    onehot = lanes == tgt_ref[...]
    picked = jnp.sum(jnp.where(onehot, block, 0.0), axis=1, keepdims=True)
    part = jnp.sum(logz - picked)

    @pl.when(i == 0)
    def _init():
        acc_ref[...] = jnp.zeros((1, 1), jnp.float32)

    acc_ref[...] += part


def kernel(idx, targets, table):
    B, T = idx.shape
    V, C = table.shape
    n_tok = B * T
    idx_flat = idx.reshape(n_tok).astype(jnp.int32)
    tgt_flat = targets.reshape(n_tok).astype(jnp.int32)

    info = plsc.get_sparse_core_info()
    nc, ns = info.num_cores, info.num_subcores
    nw = nc * ns
    bpw = n_tok // nw
    nchunk = bpw // _K

    idx3d = idx_flat.reshape(nw, nchunk, _K)

    sc_call = pl.kernel(
        functools.partial(_sc_gather_kernel, nc, bpw, nchunk),
        out_type=jax.ShapeDtypeStruct((n_tok, C), jnp.float32),
        mesh=plsc.VectorSubcoreMesh(
            core_axis_name="c", subcore_axis_name="s"
        ),
        scratch_types=[
            pltpu.VMEM((nchunk, _K), jnp.int32),
            pltpu.VMEM((_NBUF, _K, C), jnp.float32),
            pltpu.SemaphoreType.DMA((_NBUF,)),
            pltpu.SemaphoreType.DMA((_NBUF,)),
        ],
    )
    logits_flat = sc_call(table, idx3d)

    loss_sum = pl.pallas_call(
        _tc_loss_kernel,
        grid=(n_tok // _BLK,),
        in_specs=[
            pl.BlockSpec((_BLK, C), lambda i: (i, 0)),
            pl.BlockSpec((_BLK, 1), lambda i: (i, 0)),
        ],
        out_specs=pl.BlockSpec((1, 1), lambda i: (0, 0)),
        out_shape=jax.ShapeDtypeStruct((1, 1), jnp.float32),
    )(logits_flat, tgt_flat.reshape(n_tok, 1))

    loss = loss_sum[0, 0] / n_tok
    return logits_flat.reshape(B, T, C), loss
